# Initial kernel scaffold; baseline (speedup 1.0000x reference)
import jax
import jax.numpy as jnp
from jax.experimental import pallas as pl


def kernel(grid, color_embed, pos_embed):
    raise NotImplementedError("write your pallas kernel here")


# speedup vs baseline: 3.9826x; 3.9826x over previous
import jax
import jax.numpy as jnp
from jax.experimental import pallas as pl
from jax.experimental.pallas import tpu as pltpu

_HIDDEN = 128
_NCOLORS = 10
_KPAD = 16
_HW = 900
_BB = 8


def _embed_block(grid_ref, tab_ref, pos_ref, out_ref):
    g = grid_ref[...]
    oh = (g[:, :, None] == jax.lax.broadcasted_iota(
        jnp.int32, (_BB, _HW, _KPAD), 2)).astype(jnp.float32)
    oh = oh.reshape(_BB * _HW, _KPAD)
    x = jnp.dot(oh, tab_ref[...], preferred_element_type=jnp.float32)
    pos = jnp.broadcast_to(pos_ref[...][None], (_BB, _HW, _HIDDEN))
    out_ref[...] = x + pos.reshape(_BB * _HW, _HIDDEN)


def kernel(grid, color_embed, pos_embed):
    b, h, w = grid.shape
    hw = h * w
    g2 = grid.reshape(b, hw).astype(jnp.int32)
    tab = jnp.zeros((_KPAD, _HIDDEN), jnp.float32).at[:_NCOLORS].set(color_embed)
    pos2 = pos_embed[:h, :w].reshape(hw, _HIDDEN)
    out = pl.pallas_call(
        _embed_block,
        grid=(b // _BB,),
        in_specs=[
            pl.BlockSpec((_BB, hw), lambda i: (i, 0)),
            pl.BlockSpec((_KPAD, _HIDDEN), lambda i: (0, 0)),
            pl.BlockSpec((hw, _HIDDEN), lambda i: (0, 0)),
        ],
        out_specs=pl.BlockSpec((_BB * hw, _HIDDEN), lambda i: (i, 0)),
        out_shape=jax.ShapeDtypeStruct((b * hw, _HIDDEN), jnp.float32),
    )(g2, tab, pos2)
    return out.reshape(b, h, w, _HIDDEN)


# speedup vs baseline: 5.2282x; 1.3128x over previous
import jax
import jax.numpy as jnp
from jax.experimental import pallas as pl
from jax.experimental.pallas import tpu as pltpu

_HIDDEN = 128
_NCOLORS = 10
_KPAD = 16
_BB = 8


def _embed_block(grid_ref, tab_ref, pos_ref, out_ref):
    bb, h, w = grid_ref.shape
    g = grid_ref[...]
    oh = (g[..., None] == jax.lax.broadcasted_iota(
        jnp.int32, (bb, h, w, _KPAD), 3)).astype(jnp.float32)
    x = jnp.dot(oh.reshape(bb * h * w, _KPAD), tab_ref[...],
                preferred_element_type=jnp.float32)
    out_ref[...] = x.reshape(bb, h, w, _HIDDEN) + pos_ref[...][None]


def kernel(grid, color_embed, pos_embed):
    b, h, w = grid.shape
    g = grid.astype(jnp.int32)
    tab = jnp.zeros((_KPAD, _HIDDEN), jnp.float32).at[:_NCOLORS].set(color_embed)
    pos = pos_embed[:h, :w]
    return pl.pallas_call(
        _embed_block,
        grid=(b // _BB,),
        in_specs=[
            pl.BlockSpec((_BB, h, w), lambda i: (i, 0, 0)),
            pl.BlockSpec((_KPAD, _HIDDEN), lambda i: (0, 0)),
            pl.BlockSpec((h, w, _HIDDEN), lambda i: (0, 0, 0)),
        ],
        out_specs=pl.BlockSpec((_BB, h, w, _HIDDEN), lambda i: (i, 0, 0, 0)),
        out_shape=jax.ShapeDtypeStruct((b, h, w, _HIDDEN), jnp.float32),
    )(g, tab, pos)


# speedup vs baseline: 7.1394x; 1.3656x over previous
import jax
import jax.numpy as jnp
from jax.experimental import pallas as pl
from jax.experimental.pallas import tpu as pltpu

_HIDDEN = 128
_NCOLORS = 10
_KPAD = 16
_BB = 8


def _embed_block(grid_ref, tab_ref, pos_ref, out_ref):
    bb, h, w = grid_ref.shape
    wp = (w + 7) // 8 * 8
    g = grid_ref[...]
    gp = jnp.pad(g, ((0, 0), (0, 0), (0, wp - w)), constant_values=_KPAD - 1)
    oh = (gp[..., None] == jax.lax.broadcasted_iota(
        jnp.int32, (bb, h, wp, _KPAD), 3)).astype(jnp.float32)
    x = jnp.dot(oh.reshape(bb * h * wp, _KPAD), tab_ref[...],
                preferred_element_type=jnp.float32)
    x4 = x.reshape(bb, h, wp, _HIDDEN)[:, :, :w, :]
    out_ref[...] = x4 + pos_ref[...][None]


def kernel(grid, color_embed, pos_embed):
    b, h, w = grid.shape
    g = grid.astype(jnp.int32)
    tab = jnp.zeros((_KPAD, _HIDDEN), jnp.float32).at[:_NCOLORS].set(color_embed)
    pos = pos_embed[:h, :w]
    return pl.pallas_call(
        _embed_block,
        grid=(b // _BB,),
        in_specs=[
            pl.BlockSpec((_BB, h, w), lambda i: (i, 0, 0)),
            pl.BlockSpec((_KPAD, _HIDDEN), lambda i: (0, 0)),
            pl.BlockSpec((h, w, _HIDDEN), lambda i: (0, 0, 0)),
        ],
        out_specs=pl.BlockSpec((_BB, h, w, _HIDDEN), lambda i: (i, 0, 0, 0)),
        out_shape=jax.ShapeDtypeStruct((b, h, w, _HIDDEN), jnp.float32),
    )(g, tab, pos)
